# Initial kernel scaffold; baseline (speedup 1.0000x reference)
import jax
import jax.numpy as jnp
from jax.experimental import pallas as pl


def kernel(x, edge_index, W1, a_src1, a_dst1, b1, W2, a_src2, a_dst2, b2, W3, a_src3, a_dst3, b3, Wp, bp):
    raise NotImplementedError("write your pallas kernel here")


# speedup vs baseline: 33.0949x; 33.0949x over previous
import functools

import jax
import jax.numpy as jnp
from jax import lax
from jax.experimental import pallas as pl
from jax.experimental.pallas import tpu as pltpu
from jax.experimental.pallas import tpu_sc as plsc

N = 10000
E = 320000
D = 128
NP = 10240
NC = 2
NS = 16
L = 16
NW = NC * NS
KB = 64
EW_CH = 162
BLK = 18
NB = EW_CH // BLK
BLKE = BLK * KB
EW = EW_CH * KB
EP = EW * NW
RPT = NP // NS

_EPS = 1e-16


def _proj_math(x, w, a_s, a_d, h_ref, hs_ref, hd_ref, m_ref):
    h = jnp.dot(x, w, preferred_element_type=jnp.float32)
    h_ref[...] = h
    hsv = jnp.dot(h, a_s, preferred_element_type=jnp.float32)
    hdv = jnp.dot(h, a_d, preferred_element_type=jnp.float32)
    hs_ref[...] = hsv
    hd_ref[...] = hdv
    mval = jnp.maximum(jnp.max(hsv) + jnp.max(hdv), 0.0)
    m_ref[...] = jnp.full((1, 128), mval, jnp.float32)


def _proj_body(x, w, a_s, a_d, h_ref, hs_ref, hd_ref, m_ref):
    _proj_math(x[...], w[...], a_s[...], a_d[...], h_ref, hs_ref, hd_ref, m_ref)


_PROJ_OUT = [
    jax.ShapeDtypeStruct((NP, D), jnp.float32),
    jax.ShapeDtypeStruct((NP, 1), jnp.float32),
    jax.ShapeDtypeStruct((NP, 1), jnp.float32),
    jax.ShapeDtypeStruct((1, 128), jnp.float32),
]


def _proj(x, w, a_s, a_d):
    return pl.pallas_call(_proj_body, out_shape=_PROJ_OUT)(x, w, a_s, a_d)


def _comb_proj_body(p, dn, b, w, a_s, a_d, h_ref, hs_ref, hd_ref, m_ref):
    s = p[0] + p[1]
    denom = jnp.sum(dn[...], axis=0, keepdims=True)
    x = s / (denom.T + _EPS) + b[...]
    x = jnp.maximum(x, 0.0)
    _proj_math(x, w[...], a_s[...], a_d[...], h_ref, hs_ref, hd_ref, m_ref)


def _comb_proj(p, dn, b, w, a_s, a_d):
    return pl.pallas_call(_comb_proj_body, out_shape=_PROJ_OUT)(
        p, dn, b, w, a_s, a_d)


def _readout_body(p, dn, b, wp, bp, out):
    s = p[0] + p[1]
    denom = jnp.sum(dn[...], axis=0, keepdims=True)
    x = s / (denom.T + _EPS) + b[...]
    row = lax.broadcasted_iota(jnp.int32, (NP, D), 0)
    x = jnp.where(row < N, x, 0.0)
    g = jnp.sum(x, axis=0, keepdims=True) / N
    out[...] = jnp.dot(g, wp[...], preferred_element_type=jnp.float32) + bp[...]


def _readout(p, dn, b, wp, bp):
    return pl.pallas_call(
        _readout_body,
        out_shape=jax.ShapeDtypeStruct((1, 1), jnp.float32),
    )(p, dn, b, wp, bp)


_MESH = plsc.VectorSubcoreMesh(
    core_axis_name="c", subcore_axis_name="s", num_cores=NC, num_subcores=NS
)


@functools.partial(
    pl.kernel,
    out_type=[
        jax.ShapeDtypeStruct((NC, NP, D), jnp.float32),
        jax.ShapeDtypeStruct((NW, NP), jnp.float32),
    ],
    mesh=_MESH,
    compiler_params=pltpu.CompilerParams(needs_layout_passes=False),
    scratch_types=[
        pltpu.VMEM((NP,), jnp.float32),
        pltpu.VMEM((NP,), jnp.float32),
        pltpu.VMEM((L,), jnp.float32),
        pltpu.VMEM((BLKE,), jnp.int32),
        pltpu.VMEM((BLKE,), jnp.int32),
        pltpu.VMEM((KB,), jnp.int32),
        pltpu.VMEM((KB,), jnp.float32),
        pltpu.VMEM((KB, D), jnp.float32),
        pltpu.VMEM((NP,), jnp.float32),
        pltpu.VMEM_SHARED((NP, D), jnp.float32),
        pltpu.SemaphoreType.DMA,
        pltpu.SemaphoreType.DMA,
    ],
)
def _edge_kernel(h, hs, hd, mvec, src1, dst1, outp, denp,
                 hs_v, hd_v, m_v, src_bv, dst_bv, dst_c2, ex_v, rows_v, den_v,
                 out_sh, gsem, ssem):
    cid = lax.axis_index("c")
    sid = lax.axis_index("s")
    wid = sid * NC + cid

    pltpu.sync_copy(hs, hs_v)
    pltpu.sync_copy(hd, hd_v)
    pltpu.sync_copy(mvec, m_v)

    zero16 = jnp.zeros((L,), jnp.float32)

    def _zr(i, carry):
        for j in range(D // L):
            rows_v[i, pl.ds(j * L, L)] = zero16
        return carry

    lax.fori_loop(0, KB, _zr, 0)

    def _zd(i, carry):
        den_v[pl.ds(i * L, L)] = zero16
        return carry

    lax.fori_loop(0, NP // L, _zd, 0)

    def _zs(i, carry):
        pltpu.sync_copy(rows_v, out_sh.at[pl.ds(sid * RPT + i * KB, KB)])
        return carry

    lax.fori_loop(0, RPT // KB, _zs, 0)
    plsc.subcore_barrier()

    def _block(bi, carry):
        base = wid * EW + bi * BLKE
        pltpu.sync_copy(src1.at[pl.ds(base, BLKE)], src_bv)
        pltpu.sync_copy(dst1.at[pl.ds(base, BLKE)], dst_bv)

        def _chunk(c, carry2):
            cp = pltpu.async_copy(h.at[src_bv.at[pl.ds(c * KB, KB)]],
                                  rows_v, gsem)
            mv = m_v[...]
            for j in range(KB // L):
                s16 = src_bv[pl.ds(c * KB + j * L, L)]
                d16 = dst_bv[pl.ds(c * KB + j * L, L)]
                e = plsc.load_gather(hs_v, [s16]) + plsc.load_gather(hd_v, [d16])
                e = jnp.where(e > 0, e, 0.2 * e)
                ex = jnp.exp(e - mv)
                ex_v[pl.ds(j * L, L)] = ex
                dst_c2[pl.ds(j * L, L)] = d16
                plsc.addupdate_scatter(den_v, [d16], ex)
            cp.wait()

            def _scale(r16, inner):
                exv = ex_v[pl.ds(r16 * L, L)]
                for i in range(L):
                    sc = exv[i]
                    r = r16 * L + i
                    for j in range(D // L):
                        rows_v[r, pl.ds(j * L, L)] = (
                            rows_v[r, pl.ds(j * L, L)] * sc)
                return inner

            lax.fori_loop(0, KB // L, _scale, 0)
            pltpu.async_copy(rows_v, out_sh.at[dst_c2], ssem,
                             add=True).wait()
            return carry2

        lax.fori_loop(0, BLK, _chunk, 0)
        return carry

    lax.fori_loop(0, NB, _block, 0)

    pltpu.sync_copy(den_v, denp.at[wid])

    plsc.subcore_barrier()
    pltpu.sync_copy(out_sh.at[pl.ds(sid * RPT, RPT)],
                    outp.at[cid, pl.ds(sid * RPT, RPT)])


def kernel(x, edge_index, W1, a_src1, a_dst1, b1, W2, a_src2, a_dst2, b2,
           W3, a_src3, a_dst3, b3, Wp, bp):
    loops = jnp.arange(N, dtype=edge_index.dtype)
    src = jnp.concatenate([edge_index[0], loops])
    dst = jnp.concatenate([edge_index[1], loops])
    padn = EP - (E + N)
    pad_idx = (jnp.arange(padn, dtype=jnp.int32) % L) + N
    src1 = jnp.concatenate([src, pad_idx])
    dst1 = jnp.concatenate([dst, pad_idx])

    x_p = jnp.concatenate([x, jnp.zeros((NP - N, D), x.dtype)])

    h, hs, hd, m = _proj(x_p, W1, a_src1.reshape(D, 1), a_dst1.reshape(D, 1))
    p, dn = _edge_kernel(h, hs.reshape(NP), hd.reshape(NP), m[0, :L],
                         src1, dst1)

    for (w, a_s, a_d, b_prev) in ((W2, a_src2, a_dst2, b1),
                                  (W3, a_src3, a_dst3, b2)):
        h, hs, hd, m = _comb_proj(p, dn, b_prev.reshape(1, D), w,
                                  a_s.reshape(D, 1), a_d.reshape(D, 1))
        p, dn = _edge_kernel(h, hs.reshape(NP), hd.reshape(NP), m[0, :L],
                             src1, dst1)

    out = _readout(p, dn, b3.reshape(1, D), Wp, bp.reshape(1, 1))
    return out.reshape(1)


# speedup vs baseline: 43.6691x; 1.3195x over previous
import functools

import jax
import jax.numpy as jnp
from jax import lax
from jax.experimental import pallas as pl
from jax.experimental.pallas import tpu as pltpu
from jax.experimental.pallas import tpu_sc as plsc

N = 10000
E = 320000
D = 128
NP = 10240
NC = 2
NS = 16
L = 16
NW = NC * NS
KB = 32
EW_CH = 324
BLK = 18
NB = EW_CH // BLK
BLKE = BLK * KB
EW = EW_CH * KB
EP = EW * NW
RPT = NP // NS

_EPS = 1e-16


def _proj_math(x, w, a_s, a_d, h_ref, hs_ref, hd_ref, m_ref):
    h = jnp.dot(x, w, preferred_element_type=jnp.float32)
    h_ref[...] = h
    hsv = jnp.dot(h, a_s, preferred_element_type=jnp.float32)
    hdv = jnp.dot(h, a_d, preferred_element_type=jnp.float32)
    hs_ref[...] = hsv
    hd_ref[...] = hdv
    mval = jnp.maximum(jnp.max(hsv) + jnp.max(hdv), 0.0)
    m_ref[...] = jnp.full((1, 128), mval, jnp.float32)


def _proj_body(x, w, a_s, a_d, h_ref, hs_ref, hd_ref, m_ref):
    _proj_math(x[...], w[...], a_s[...], a_d[...], h_ref, hs_ref, hd_ref, m_ref)


_PROJ_OUT = [
    jax.ShapeDtypeStruct((NP, D), jnp.float32),
    jax.ShapeDtypeStruct((NP, 1), jnp.float32),
    jax.ShapeDtypeStruct((NP, 1), jnp.float32),
    jax.ShapeDtypeStruct((1, 128), jnp.float32),
]


def _proj(x, w, a_s, a_d):
    return pl.pallas_call(_proj_body, out_shape=_PROJ_OUT)(x, w, a_s, a_d)


def _comb_proj_body(p, dn, b, w, a_s, a_d, h_ref, hs_ref, hd_ref, m_ref):
    s = p[0] + p[1]
    denom = jnp.sum(dn[...], axis=0, keepdims=True)
    x = s / (denom.T + _EPS) + b[...]
    x = jnp.maximum(x, 0.0)
    _proj_math(x, w[...], a_s[...], a_d[...], h_ref, hs_ref, hd_ref, m_ref)


def _comb_proj(p, dn, b, w, a_s, a_d):
    return pl.pallas_call(_comb_proj_body, out_shape=_PROJ_OUT)(
        p, dn, b, w, a_s, a_d)


def _readout_body(p, dn, b, wp, bp, out):
    s = p[0] + p[1]
    denom = jnp.sum(dn[...], axis=0, keepdims=True)
    x = s / (denom.T + _EPS) + b[...]
    row = lax.broadcasted_iota(jnp.int32, (NP, D), 0)
    x = jnp.where(row < N, x, 0.0)
    g = jnp.sum(x, axis=0, keepdims=True) / N
    out[...] = jnp.dot(g, wp[...], preferred_element_type=jnp.float32) + bp[...]


def _readout(p, dn, b, wp, bp):
    return pl.pallas_call(
        _readout_body,
        out_shape=jax.ShapeDtypeStruct((1, 1), jnp.float32),
    )(p, dn, b, wp, bp)


_MESH = plsc.VectorSubcoreMesh(
    core_axis_name="c", subcore_axis_name="s", num_cores=NC, num_subcores=NS
)


@functools.partial(
    pl.kernel,
    out_type=[
        jax.ShapeDtypeStruct((NC, NP, D), jnp.float32),
        jax.ShapeDtypeStruct((NW, NP), jnp.float32),
    ],
    mesh=_MESH,
    compiler_params=pltpu.CompilerParams(needs_layout_passes=False),
    scratch_types=[
        pltpu.VMEM((NP,), jnp.float32),
        pltpu.VMEM((NP,), jnp.float32),
        pltpu.VMEM((L,), jnp.float32),
        pltpu.VMEM((BLKE,), jnp.int32),
        pltpu.VMEM((BLKE,), jnp.int32),
        [pltpu.VMEM((KB,), jnp.int32) for _ in range(3)],
        pltpu.VMEM((KB,), jnp.float32),
        [pltpu.VMEM((KB, D), jnp.float32) for _ in range(3)],
        pltpu.VMEM((NP,), jnp.float32),
        pltpu.VMEM_SHARED((NP, D), jnp.float32),
        [pltpu.SemaphoreType.DMA for _ in range(3)],
        [pltpu.SemaphoreType.DMA for _ in range(3)],
    ],
)
def _edge_kernel(h, hs, hd, mvec, src1, dst1, outp, denp,
                 hs_v, hd_v, m_v, src_bv, dst_bv, dst_c2, ex_v, rows,
                 den_v, out_sh, gsem, ssem):
    cid = lax.axis_index("c")
    sid = lax.axis_index("s")
    wid = sid * NC + cid

    pltpu.sync_copy(hs, hs_v)
    pltpu.sync_copy(hd, hd_v)
    pltpu.sync_copy(mvec, m_v)

    zero16 = jnp.zeros((L,), jnp.float32)

    def _zr(i, carry):
        for j in range(D // L):
            rows[0][i, pl.ds(j * L, L)] = zero16
        return carry

    lax.fori_loop(0, KB, _zr, 0)

    def _zd(i, carry):
        den_v[pl.ds(i * L, L)] = zero16
        return carry

    lax.fori_loop(0, NP // L, _zd, 0)

    def _zs(i, carry):
        pltpu.sync_copy(rows[0], out_sh.at[pl.ds(sid * RPT + i * KB, KB)])
        return carry

    lax.fori_loop(0, RPT // KB, _zs, 0)
    plsc.subcore_barrier()

    def _g_desc(c, b):
        return pltpu.make_async_copy(
            h.at[src_bv.at[pl.ds(c * KB, KB)]], rows[b], gsem[b])

    def _g_start(c, b):
        pltpu.async_copy(h.at[src_bv.at[pl.ds(c * KB, KB)]], rows[b], gsem[b])

    def _s_desc(b):
        return pltpu.make_async_copy(rows[b], out_sh.at[dst_c2[b]], ssem[b])

    def _s_start(b):
        pltpu.async_copy(rows[b], out_sh.at[dst_c2[b]], ssem[b], add=True)

    def _do_chunk(c, b):
        _g_desc(c, b).wait()
        mv = m_v[...]
        for j in range(KB // L):
            s16 = src_bv[pl.ds(c * KB + j * L, L)]
            d16 = dst_bv[pl.ds(c * KB + j * L, L)]
            e = plsc.load_gather(hs_v, [s16]) + plsc.load_gather(hd_v, [d16])
            e = jnp.where(e > 0, e, 0.2 * e)
            ex = jnp.exp(e - mv)
            ex_v[pl.ds(j * L, L)] = ex
            dst_c2[b][pl.ds(j * L, L)] = d16
            plsc.addupdate_scatter(den_v, [d16], ex)

        def _scale(r16, inner):
            exv = ex_v[pl.ds(r16 * L, L)]
            for i in range(L):
                sc = exv[i]
                r = r16 * L + i
                for j in range(D // L):
                    rows[b][r, pl.ds(j * L, L)] = (
                        rows[b][r, pl.ds(j * L, L)] * sc)
            return inner

        lax.fori_loop(0, KB // L, _scale, 0)

    def _block(bi, carry):
        base = wid * EW + bi * BLKE
        pltpu.sync_copy(src1.at[pl.ds(base, BLKE)], src_bv)
        pltpu.sync_copy(dst1.at[pl.ds(base, BLKE)], dst_bv)
        _g_start(0, 0)
        _g_start(1, 1)

        def _group(g, carry2):
            for k in range(3):
                c = g * 3 + k
                _do_chunk(c, k)
                nb = (k + 2) % 3

                @pl.when(c >= 1)
                def _():
                    _s_desc(nb).wait()

                @pl.when(c + 2 < BLK)
                def _():
                    _g_start(c + 2, nb)

                _s_start(k)
            return carry2

        lax.fori_loop(0, BLK // 3, _group, 0)
        _s_desc(2).wait()
        return carry

    lax.fori_loop(0, NB, _block, 0)

    pltpu.sync_copy(den_v, denp.at[wid])

    plsc.subcore_barrier()
    pltpu.sync_copy(out_sh.at[pl.ds(sid * RPT, RPT)],
                    outp.at[cid, pl.ds(sid * RPT, RPT)])


def kernel(x, edge_index, W1, a_src1, a_dst1, b1, W2, a_src2, a_dst2, b2,
           W3, a_src3, a_dst3, b3, Wp, bp):
    loops = jnp.arange(N, dtype=edge_index.dtype)
    src = jnp.concatenate([edge_index[0], loops])
    dst = jnp.concatenate([edge_index[1], loops])
    padn = EP - (E + N)
    pad_idx = (jnp.arange(padn, dtype=jnp.int32) % L) + N
    src1 = jnp.concatenate([src, pad_idx])
    dst1 = jnp.concatenate([dst, pad_idx])

    x_p = jnp.concatenate([x, jnp.zeros((NP - N, D), x.dtype)])

    h, hs, hd, m = _proj(x_p, W1, a_src1.reshape(D, 1), a_dst1.reshape(D, 1))
    p, dn = _edge_kernel(h, hs.reshape(NP), hd.reshape(NP), m[0, :L],
                         src1, dst1)

    for (w, a_s, a_d, b_prev) in ((W2, a_src2, a_dst2, b1),
                                  (W3, a_src3, a_dst3, b2)):
        h, hs, hd, m = _comb_proj(p, dn, b_prev.reshape(1, D), w,
                                  a_s.reshape(D, 1), a_d.reshape(D, 1))
        p, dn = _edge_kernel(h, hs.reshape(NP), hd.reshape(NP), m[0, :L],
                             src1, dst1)

    out = _readout(p, dn, b3.reshape(1, D), Wp, bp.reshape(1, 1))
    return out.reshape(1)


# speedup vs baseline: 50.9427x; 1.1666x over previous
import functools

import jax
import jax.numpy as jnp
from jax import lax
from jax.experimental import pallas as pl
from jax.experimental.pallas import tpu as pltpu
from jax.experimental.pallas import tpu_sc as plsc

N = 10000
E = 320000
D = 128
NP = 10240
NC = 2
NS = 16
L = 16
NW = NC * NS
KB = 32
EW_CH = 324
BLK = 18
NB = EW_CH // BLK
BLKE = BLK * KB
EW = EW_CH * KB
EP = EW * NW
RPT = NP // NS

_EPS = 1e-16


def _proj_math(x, w, a_s, a_d, h_ref, hs_ref, hd_ref, m_ref):
    h = jnp.dot(x, w, preferred_element_type=jnp.float32)
    h_ref[...] = h
    hsv = jnp.dot(h, a_s, preferred_element_type=jnp.float32)
    hdv = jnp.dot(h, a_d, preferred_element_type=jnp.float32)
    hs_ref[...] = hsv
    hd_ref[...] = hdv
    mval = jnp.maximum(jnp.max(hsv) + jnp.max(hdv), 0.0)
    m_ref[...] = jnp.full((1, 128), mval, jnp.float32)


def _proj_body(x, w, a_s, a_d, h_ref, hs_ref, hd_ref, m_ref):
    _proj_math(x[...], w[...], a_s[...], a_d[...], h_ref, hs_ref, hd_ref, m_ref)


_PROJ_OUT = [
    jax.ShapeDtypeStruct((NP, D), jnp.float32),
    jax.ShapeDtypeStruct((NP, 1), jnp.float32),
    jax.ShapeDtypeStruct((NP, 1), jnp.float32),
    jax.ShapeDtypeStruct((1, 128), jnp.float32),
]


def _proj(x, w, a_s, a_d):
    return pl.pallas_call(_proj_body, out_shape=_PROJ_OUT)(x, w, a_s, a_d)


def _comb_proj_body(p, dn, b, w, a_s, a_d, h_ref, hs_ref, hd_ref, m_ref):
    s = p[0] + p[1]
    denom = jnp.sum(dn[...], axis=0, keepdims=True)
    x = s / (denom.T + _EPS) + b[...]
    x = jnp.maximum(x, 0.0)
    _proj_math(x, w[...], a_s[...], a_d[...], h_ref, hs_ref, hd_ref, m_ref)


def _comb_proj(p, dn, b, w, a_s, a_d):
    return pl.pallas_call(_comb_proj_body, out_shape=_PROJ_OUT)(
        p, dn, b, w, a_s, a_d)


def _readout_body(p, dn, b, wp, bp, out):
    s = p[0] + p[1]
    denom = jnp.sum(dn[...], axis=0, keepdims=True)
    x = s / (denom.T + _EPS) + b[...]
    row = lax.broadcasted_iota(jnp.int32, (NP, D), 0)
    x = jnp.where(row < N, x, 0.0)
    g = jnp.sum(x, axis=0, keepdims=True) / N
    out[...] = jnp.dot(g, wp[...], preferred_element_type=jnp.float32) + bp[...]


def _readout(p, dn, b, wp, bp):
    return pl.pallas_call(
        _readout_body,
        out_shape=jax.ShapeDtypeStruct((1, 1), jnp.float32),
    )(p, dn, b, wp, bp)


_MESH = plsc.VectorSubcoreMesh(
    core_axis_name="c", subcore_axis_name="s", num_cores=NC, num_subcores=NS
)


@functools.partial(
    pl.kernel,
    out_type=[
        jax.ShapeDtypeStruct((NC, NP, D), jnp.float32),
        jax.ShapeDtypeStruct((NW, NP), jnp.float32),
    ],
    mesh=_MESH,
    compiler_params=pltpu.CompilerParams(needs_layout_passes=False),
    scratch_types=[
        pltpu.VMEM((NP,), jnp.float32),
        pltpu.VMEM((NP,), jnp.float32),
        pltpu.VMEM((L,), jnp.float32),
        pltpu.VMEM((BLKE,), jnp.int32),
        pltpu.VMEM((BLKE,), jnp.int32),
        [pltpu.VMEM((KB,), jnp.int32) for _ in range(3)],
        pltpu.VMEM((KB,), jnp.float32),
        [pltpu.VMEM((KB, D), jnp.float32) for _ in range(3)],
        pltpu.VMEM((NP,), jnp.float32),
        pltpu.VMEM_SHARED((NP, D), jnp.float32),
        [pltpu.SemaphoreType.DMA for _ in range(3)],
        [pltpu.SemaphoreType.DMA for _ in range(3)],
        [pltpu.SemaphoreType.DMA for _ in range(2)],
    ],
)
def _edge_kernel(h, hs, hd, mvec, src1, dst1, outp, denp,
                 hs_v, hd_v, m_v, src_bv, dst_bv, dst_c2, ex_v, rows,
                 den_v, out_sh, gsem, ssem, isem):
    cid = lax.axis_index("c")
    sid = lax.axis_index("s")
    wid = sid * NC + cid

    pltpu.sync_copy(hs, hs_v)
    pltpu.sync_copy(hd, hd_v)
    pltpu.sync_copy(mvec, m_v)

    zero16 = jnp.zeros((L,), jnp.float32)

    def _zr(i, carry):
        for j in range(D // L):
            rows[0][i, pl.ds(j * L, L)] = zero16
        return carry

    lax.fori_loop(0, KB, _zr, 0)

    def _zd(i, carry):
        den_v[pl.ds(i * L, L)] = zero16
        return carry

    lax.fori_loop(0, NP // L, _zd, 0)

    def _zs(i, carry):
        pltpu.sync_copy(rows[0], out_sh.at[pl.ds(sid * RPT + i * KB, KB)])
        return carry

    lax.fori_loop(0, RPT // KB, _zs, 0)
    plsc.subcore_barrier()

    def _g_desc(c, b):
        return pltpu.make_async_copy(
            h.at[src_bv.at[pl.ds(c * KB, KB)]], rows[b], gsem[b])

    def _g_start(c, b):
        pltpu.async_copy(h.at[src_bv.at[pl.ds(c * KB, KB)]], rows[b], gsem[b])

    def _s_desc(b):
        return pltpu.make_async_copy(rows[b], out_sh.at[dst_c2[b]], ssem[b])

    def _s_start(b):
        pltpu.async_copy(rows[b], out_sh.at[dst_c2[b]], ssem[b], add=True)

    def _ex_chunk(c, b):
        mv = m_v[...]
        for j in range(KB // L):
            s16 = src_bv[pl.ds(c * KB + j * L, L)]
            d16 = dst_bv[pl.ds(c * KB + j * L, L)]
            e = plsc.load_gather(hs_v, [s16]) + plsc.load_gather(hd_v, [d16])
            e = jnp.where(e > 0, e, 0.2 * e)
            ex = jnp.exp(e - mv)
            ex_v[pl.ds(j * L, L)] = ex
            dst_c2[b][pl.ds(j * L, L)] = d16
            plsc.addupdate_scatter(den_v, [d16], ex)

    def _scale_chunk(b):
        def _scale(r16):
            exv = ex_v[pl.ds(r16 * L, L)]
            for i in range(L):
                sc = exv[i]
                r = r16 * L + i
                for j in range(D // L):
                    rows[b][r, pl.ds(j * L, L)] = (
                        rows[b][r, pl.ds(j * L, L)] * sc)

        plsc.parallel_loop(0, KB // L, unroll=2)(_scale)

    def _block(bi, carry):
        base = wid * EW + bi * BLKE
        pltpu.async_copy(src1.at[pl.ds(base, BLKE)], src_bv, isem[0])
        pltpu.async_copy(dst1.at[pl.ds(base, BLKE)], dst_bv, isem[1])
        pltpu.make_async_copy(src1.at[pl.ds(base, BLKE)], src_bv,
                              isem[0]).wait()
        pltpu.make_async_copy(dst1.at[pl.ds(base, BLKE)], dst_bv,
                              isem[1]).wait()
        _g_start(0, 0)
        _g_start(1, 1)

        def _group(g, carry2):
            for k in range(3):
                c = g * 3 + k
                nb = (k + 2) % 3
                _ex_chunk(c, k)

                @pl.when(c >= 1)
                def _():
                    _s_desc(nb).wait()

                @pl.when(c + 2 < BLK)
                def _():
                    _g_start(c + 2, nb)

                _g_desc(c, k).wait()
                _scale_chunk(k)
                _s_start(k)
            return carry2

        lax.fori_loop(0, BLK // 3, _group, 0)
        _s_desc(2).wait()
        return carry

    lax.fori_loop(0, NB, _block, 0)

    pltpu.sync_copy(den_v, denp.at[wid])

    plsc.subcore_barrier()
    pltpu.sync_copy(out_sh.at[pl.ds(sid * RPT, RPT)],
                    outp.at[cid, pl.ds(sid * RPT, RPT)])


def kernel(x, edge_index, W1, a_src1, a_dst1, b1, W2, a_src2, a_dst2, b2,
           W3, a_src3, a_dst3, b3, Wp, bp):
    loops = jnp.arange(N, dtype=edge_index.dtype)
    src = jnp.concatenate([edge_index[0], loops])
    dst = jnp.concatenate([edge_index[1], loops])
    padn = EP - (E + N)
    pad_idx = (jnp.arange(padn, dtype=jnp.int32) % L) + N
    src1 = jnp.concatenate([src, pad_idx])
    dst1 = jnp.concatenate([dst, pad_idx])

    x_p = jnp.concatenate([x, jnp.zeros((NP - N, D), x.dtype)])

    h, hs, hd, m = _proj(x_p, W1, a_src1.reshape(D, 1), a_dst1.reshape(D, 1))
    p, dn = _edge_kernel(h, hs.reshape(NP), hd.reshape(NP), m[0, :L],
                         src1, dst1)

    for (w, a_s, a_d, b_prev) in ((W2, a_src2, a_dst2, b1),
                                  (W3, a_src3, a_dst3, b2)):
        h, hs, hd, m = _comb_proj(p, dn, b_prev.reshape(1, D), w,
                                  a_s.reshape(D, 1), a_d.reshape(D, 1))
        p, dn = _edge_kernel(h, hs.reshape(NP), hd.reshape(NP), m[0, :L],
                             src1, dst1)

    out = _readout(p, dn, b3.reshape(1, D), Wp, bp.reshape(1, 1))
    return out.reshape(1)


# speedup vs baseline: 55.6905x; 1.0932x over previous
import functools

import jax
import jax.numpy as jnp
from jax import lax
from jax.experimental import pallas as pl
from jax.experimental.pallas import tpu as pltpu
from jax.experimental.pallas import tpu_sc as plsc

N = 10000
E = 320000
D = 128
NP = 10240
NC = 2
NS = 16
L = 16
NW = NC * NS
KB = 32
EW_CH = 324
BLK = 36
NB = EW_CH // BLK
BLKE = BLK * KB
EW = EW_CH * KB
EP = EW * NW
RPT = NP // NS

_EPS = 1e-16


def _proj_math(x, w, a_s, a_d, h_ref, hs_ref, hd_ref, m_ref):
    h = jnp.dot(x, w, preferred_element_type=jnp.float32)
    h_ref[...] = h
    hsv = jnp.dot(h, a_s, preferred_element_type=jnp.float32)
    hdv = jnp.dot(h, a_d, preferred_element_type=jnp.float32)
    hs_ref[...] = hsv
    hd_ref[...] = hdv
    mval = jnp.maximum(jnp.max(hsv) + jnp.max(hdv), 0.0)
    m_ref[...] = jnp.full((1, 128), mval, jnp.float32)


def _proj_body(x, w, a_s, a_d, h_ref, hs_ref, hd_ref, m_ref):
    _proj_math(x[...], w[...], a_s[...], a_d[...], h_ref, hs_ref, hd_ref, m_ref)


_PROJ_OUT = [
    jax.ShapeDtypeStruct((NP, D), jnp.float32),
    jax.ShapeDtypeStruct((NP, 1), jnp.float32),
    jax.ShapeDtypeStruct((NP, 1), jnp.float32),
    jax.ShapeDtypeStruct((1, 128), jnp.float32),
]


def _proj(x, w, a_s, a_d):
    return pl.pallas_call(_proj_body, out_shape=_PROJ_OUT)(x, w, a_s, a_d)


def _comb_proj_body(p, dn, b, w, a_s, a_d, h_ref, hs_ref, hd_ref, m_ref):
    s = p[0] + p[1]
    denom = jnp.sum(dn[...], axis=0, keepdims=True)
    x = s / (denom.T + _EPS) + b[...]
    x = jnp.maximum(x, 0.0)
    _proj_math(x, w[...], a_s[...], a_d[...], h_ref, hs_ref, hd_ref, m_ref)


def _comb_proj(p, dn, b, w, a_s, a_d):
    return pl.pallas_call(_comb_proj_body, out_shape=_PROJ_OUT)(
        p, dn, b, w, a_s, a_d)


def _readout_body(p, dn, b, wp, bp, out):
    s = p[0] + p[1]
    denom = jnp.sum(dn[...], axis=0, keepdims=True)
    x = s / (denom.T + _EPS) + b[...]
    row = lax.broadcasted_iota(jnp.int32, (NP, D), 0)
    x = jnp.where(row < N, x, 0.0)
    g = jnp.sum(x, axis=0, keepdims=True) / N
    out[...] = jnp.dot(g, wp[...], preferred_element_type=jnp.float32) + bp[...]


def _readout(p, dn, b, wp, bp):
    return pl.pallas_call(
        _readout_body,
        out_shape=jax.ShapeDtypeStruct((1, 1), jnp.float32),
    )(p, dn, b, wp, bp)


_MESH = plsc.VectorSubcoreMesh(
    core_axis_name="c", subcore_axis_name="s", num_cores=NC, num_subcores=NS
)


@functools.partial(
    pl.kernel,
    out_type=[
        jax.ShapeDtypeStruct((NC, NP, D), jnp.float32),
        jax.ShapeDtypeStruct((NW, NP), jnp.float32),
    ],
    mesh=_MESH,
    compiler_params=pltpu.CompilerParams(needs_layout_passes=False),
    scratch_types=[
        pltpu.VMEM((NP,), jnp.float32),
        pltpu.VMEM((NP,), jnp.float32),
        pltpu.VMEM((L,), jnp.float32),
        pltpu.VMEM((BLKE,), jnp.int32),
        pltpu.VMEM((BLKE,), jnp.int32),
        [pltpu.VMEM((KB,), jnp.int32) for _ in range(3)],
        pltpu.VMEM((KB,), jnp.float32),
        [pltpu.VMEM((KB, D), jnp.float32) for _ in range(3)],
        pltpu.VMEM((NP,), jnp.float32),
        pltpu.VMEM_SHARED((NP, D), jnp.float32),
        [pltpu.SemaphoreType.DMA for _ in range(3)],
        [pltpu.SemaphoreType.DMA for _ in range(3)],
        [pltpu.SemaphoreType.DMA for _ in range(2)],
    ],
)
def _edge_kernel(h, hs, hd, mvec, src1, dst1, outp, denp,
                 hs_v, hd_v, m_v, src_bv, dst_bv, dst_c2, ex_v, rows,
                 den_v, out_sh, gsem, ssem, isem):
    cid = lax.axis_index("c")
    sid = lax.axis_index("s")
    wid = sid * NC + cid

    pltpu.async_copy(hs, hs_v, isem[0])
    pltpu.async_copy(hd, hd_v, isem[1])
    pltpu.async_copy(mvec, m_v, gsem[0])

    zero16 = jnp.zeros((L,), jnp.float32)

    def _zr(i, carry):
        for j in range(D // L):
            rows[0][i, pl.ds(j * L, L)] = zero16
        return carry

    lax.fori_loop(0, KB, _zr, 0)

    def _zd(i, carry):
        den_v[pl.ds(i * L, L)] = zero16
        return carry

    lax.fori_loop(0, NP // L, _zd, 0)

    def _zs(i, carry):
        pltpu.sync_copy(rows[0], out_sh.at[pl.ds(sid * RPT + i * KB, KB)])
        return carry

    lax.fori_loop(0, RPT // KB, _zs, 0)
    pltpu.make_async_copy(hs, hs_v, isem[0]).wait()
    pltpu.make_async_copy(hd, hd_v, isem[1]).wait()
    pltpu.make_async_copy(mvec, m_v, gsem[0]).wait()
    plsc.subcore_barrier()

    def _g_desc(c, b):
        return pltpu.make_async_copy(
            h.at[src_bv.at[pl.ds(c * KB, KB)]], rows[b], gsem[b])

    def _g_start(c, b):
        pltpu.async_copy(h.at[src_bv.at[pl.ds(c * KB, KB)]], rows[b], gsem[b])

    def _s_desc(b):
        return pltpu.make_async_copy(rows[b], out_sh.at[dst_c2[b]], ssem[b])

    def _s_start(b):
        pltpu.async_copy(rows[b], out_sh.at[dst_c2[b]], ssem[b], add=True)

    def _ex_chunk(c, b):
        mv = m_v[...]
        for j in range(KB // L):
            s16 = src_bv[pl.ds(c * KB + j * L, L)]
            d16 = dst_bv[pl.ds(c * KB + j * L, L)]
            e = plsc.load_gather(hs_v, [s16]) + plsc.load_gather(hd_v, [d16])
            e = jnp.where(e > 0, e, 0.2 * e)
            ex = jnp.exp(e - mv)
            ex_v[pl.ds(j * L, L)] = ex
            dst_c2[b][pl.ds(j * L, L)] = d16
            plsc.addupdate_scatter(den_v, [d16], ex)

    def _scale_chunk(b):
        def _scale(r16):
            exv = ex_v[pl.ds(r16 * L, L)]
            for i in range(L):
                sc = exv[i]
                r = r16 * L + i
                for j in range(D // L):
                    rows[b][r, pl.ds(j * L, L)] = (
                        rows[b][r, pl.ds(j * L, L)] * sc)

        plsc.parallel_loop(0, KB // L, unroll=2)(_scale)

    def _block(bi, carry):
        base = wid * EW + bi * BLKE
        pltpu.async_copy(src1.at[pl.ds(base, BLKE)], src_bv, isem[0])
        pltpu.async_copy(dst1.at[pl.ds(base, BLKE)], dst_bv, isem[1])
        pltpu.make_async_copy(src1.at[pl.ds(base, BLKE)], src_bv,
                              isem[0]).wait()
        pltpu.make_async_copy(dst1.at[pl.ds(base, BLKE)], dst_bv,
                              isem[1]).wait()
        _g_start(0, 0)
        _g_start(1, 1)

        def _group(g, carry2):
            for k in range(3):
                c = g * 3 + k
                nb = (k + 2) % 3
                _ex_chunk(c, k)

                @pl.when(c >= 1)
                def _():
                    _s_desc(nb).wait()

                @pl.when(c + 2 < BLK)
                def _():
                    _g_start(c + 2, nb)

                _g_desc(c, k).wait()
                _scale_chunk(k)
                _s_start(k)
            return carry2

        lax.fori_loop(0, BLK // 3, _group, 0)
        _s_desc(2).wait()
        return carry

    lax.fori_loop(0, NB, _block, 0)

    pltpu.sync_copy(den_v, denp.at[wid])

    plsc.subcore_barrier()
    pltpu.sync_copy(out_sh.at[pl.ds(sid * RPT, RPT)],
                    outp.at[cid, pl.ds(sid * RPT, RPT)])


def kernel(x, edge_index, W1, a_src1, a_dst1, b1, W2, a_src2, a_dst2, b2,
           W3, a_src3, a_dst3, b3, Wp, bp):
    loops = jnp.arange(N, dtype=edge_index.dtype)
    src = jnp.concatenate([edge_index[0], loops])
    dst = jnp.concatenate([edge_index[1], loops])
    padn = EP - (E + N)
    pad_idx = (jnp.arange(padn, dtype=jnp.int32) % L) + N
    src1 = jnp.concatenate([src, pad_idx])
    dst1 = jnp.concatenate([dst, pad_idx])

    x_p = jnp.concatenate([x, jnp.zeros((NP - N, D), x.dtype)])

    h, hs, hd, m = _proj(x_p, W1, a_src1.reshape(D, 1), a_dst1.reshape(D, 1))
    p, dn = _edge_kernel(h, hs.reshape(NP), hd.reshape(NP), m[0, :L],
                         src1, dst1)

    for (w, a_s, a_d, b_prev) in ((W2, a_src2, a_dst2, b1),
                                  (W3, a_src3, a_dst3, b2)):
        h, hs, hd, m = _comb_proj(p, dn, b_prev.reshape(1, D), w,
                                  a_s.reshape(D, 1), a_d.reshape(D, 1))
        p, dn = _edge_kernel(h, hs.reshape(NP), hd.reshape(NP), m[0, :L],
                             src1, dst1)

    out = _readout(p, dn, b3.reshape(1, D), Wp, bp.reshape(1, 1))
    return out.reshape(1)
